# Initial kernel scaffold; baseline (speedup 1.0000x reference)
import jax
import jax.numpy as jnp
from jax.experimental import pallas as pl


def kernel(sequence, token_table, pos_table):
    raise NotImplementedError("write your pallas kernel here")


# speedup vs baseline: 3.2920x; 3.2920x over previous
import functools

import jax
import jax.numpy as jnp
from jax import lax
from jax.experimental import pallas as pl
from jax.experimental.pallas import tpu as pltpu
from jax.experimental.pallas import tpu_sc as plsc

VOCAB = 100000
EMBED = 64
MAX_LEN = 200
BATCH = 4096

NUM_CORES = 2
NUM_SUBCORES = 16
NUM_WORKERS = NUM_CORES * NUM_SUBCORES
SEQ_PER_W = BATCH // NUM_WORKERS
HALF = MAX_LEN // 2


def _sc_embed(seq2d, token_table, pos_table):
    mesh = plsc.VectorSubcoreMesh(core_axis_name="c", subcore_axis_name="s")

    @functools.partial(
        pl.kernel,
        mesh=mesh,
        out_type=jax.ShapeDtypeStruct((BATCH * MAX_LEN, EMBED), jnp.float32),
        scratch_types=[
            pltpu.VMEM((2 * SEQ_PER_W, HALF), jnp.int32),
            pltpu.VMEM((MAX_LEN, EMBED), jnp.float32),
            pltpu.VMEM((MAX_LEN, EMBED), jnp.float32),
            pltpu.SemaphoreType.DMA,
        ],
        compiler_params=pltpu.CompilerParams(use_tc_tiling_on_sc=False),
    )
    def k(seq_hbm, tok_hbm, pos_hbm, out_hbm, idx_v, pos_v, rows_v, sem):
        wid = lax.axis_index("s") * NUM_CORES + lax.axis_index("c")
        pltpu.sync_copy(pos_hbm, pos_v)
        pltpu.sync_copy(seq_hbm.at[pl.ds(wid * 2 * SEQ_PER_W, 2 * SEQ_PER_W)],
                        idx_v)
        out_base = wid * SEQ_PER_W * MAX_LEN

        def body(s, carry):
            g1 = pltpu.async_copy(tok_hbm.at[idx_v.at[2 * s]],
                                  rows_v.at[pl.ds(0, HALF)], sem)
            g2 = pltpu.async_copy(tok_hbm.at[idx_v.at[2 * s + 1]],
                                  rows_v.at[pl.ds(HALF, HALF)], sem)
            g1.wait()
            g2.wait()

            def add_body(i, c):
                for j in range(EMBED // 16):
                    sl = pl.ds(j * 16, 16)
                    rows_v[i, sl] = rows_v[i, sl] + pos_v[i, sl]
                return c

            lax.fori_loop(0, MAX_LEN, add_body, 0)
            pltpu.sync_copy(rows_v,
                            out_hbm.at[pl.ds(out_base + s * MAX_LEN, MAX_LEN)])
            return carry

        lax.fori_loop(0, SEQ_PER_W, body, 0)

    return k(seq2d, token_table, pos_table)


def kernel(sequence, token_table, pos_table):
    seq2d = sequence.reshape(2 * BATCH, HALF).astype(jnp.int32)
    out = _sc_embed(seq2d, token_table, pos_table)
    return out.reshape(BATCH, MAX_LEN, EMBED)


# speedup vs baseline: 3.9891x; 1.2118x over previous
import functools

import jax
import jax.numpy as jnp
from jax import lax
from jax.experimental import pallas as pl
from jax.experimental.pallas import tpu as pltpu
from jax.experimental.pallas import tpu_sc as plsc

VOCAB = 100000
EMBED = 64
MAX_LEN = 200
BATCH = 4096

NUM_CORES = 2
NUM_SUBCORES = 16
NUM_WORKERS = NUM_CORES * NUM_SUBCORES
SEQ_PER_W = BATCH // NUM_WORKERS
HALF = MAX_LEN // 2


def _sc_embed(seq2d, token_table, pos_table):
    mesh = plsc.VectorSubcoreMesh(core_axis_name="c", subcore_axis_name="s")

    @functools.partial(
        pl.kernel,
        mesh=mesh,
        out_type=jax.ShapeDtypeStruct((BATCH * MAX_LEN, EMBED), jnp.float32),
        scratch_types=[
            pltpu.VMEM((2 * SEQ_PER_W, HALF), jnp.int32),
            pltpu.VMEM((MAX_LEN, EMBED), jnp.float32),
            pltpu.VMEM((MAX_LEN, EMBED), jnp.float32),
            pltpu.VMEM((MAX_LEN, EMBED), jnp.float32),
            pltpu.SemaphoreType.DMA,
            pltpu.SemaphoreType.DMA,
        ],
        compiler_params=pltpu.CompilerParams(use_tc_tiling_on_sc=False),
    )
    def k(seq_hbm, tok_hbm, pos_hbm, out_hbm, idx_v, pos_v, buf_a, buf_b,
          sem_a, sem_b):
        wid = lax.axis_index("s") * NUM_CORES + lax.axis_index("c")
        pltpu.sync_copy(pos_hbm, pos_v)
        pltpu.sync_copy(seq_hbm.at[pl.ds(wid * 2 * SEQ_PER_W, 2 * SEQ_PER_W)],
                        idx_v)
        out_base = wid * SEQ_PER_W * MAX_LEN

        def gather(s, buf, sem):
            pltpu.async_copy(tok_hbm.at[idx_v.at[2 * s]],
                             buf.at[pl.ds(0, HALF)], sem)
            pltpu.async_copy(tok_hbm.at[idx_v.at[2 * s + 1]],
                             buf.at[pl.ds(HALF, HALF)], sem)

        def wait_gathers(buf, sem):
            pltpu.make_async_copy(tok_hbm.at[pl.ds(0, MAX_LEN)], buf, sem).wait()

        def add_pos(buf):
            def add_body(i, c):
                for j in range(EMBED // 16):
                    sl = pl.ds(j * 16, 16)
                    plsc.addupdate(buf.at[i, sl], pos_v[i, sl])
                return c

            lax.fori_loop(0, MAX_LEN, add_body, 0)

        def write_out(s, buf):
            pltpu.sync_copy(buf,
                            out_hbm.at[pl.ds(out_base + s * MAX_LEN, MAX_LEN)])

        gather(0, buf_a, sem_a)
        gather(1, buf_b, sem_b)

        def body(t, c):
            s0 = 2 * t
            s1 = 2 * t + 1
            wait_gathers(buf_a, sem_a)
            add_pos(buf_a)
            write_out(s0, buf_a)
            sn0 = jnp.where(s0 + 2 < SEQ_PER_W, s0 + 2, 0)
            gather(sn0, buf_a, sem_a)
            wait_gathers(buf_b, sem_b)
            add_pos(buf_b)
            write_out(s1, buf_b)
            sn1 = jnp.where(s1 + 2 < SEQ_PER_W, s1 + 2, 1)
            gather(sn1, buf_b, sem_b)
            return c

        lax.fori_loop(0, SEQ_PER_W // 2, body, 0)
        wait_gathers(buf_a, sem_a)
        wait_gathers(buf_b, sem_b)

    return k(seq2d, token_table, pos_table)


def kernel(sequence, token_table, pos_table):
    seq2d = sequence.reshape(2 * BATCH, HALF).astype(jnp.int32)
    out = _sc_embed(seq2d, token_table, pos_table)
    return out.reshape(BATCH, MAX_LEN, EMBED)
